# baseline (speedup 1.0000x reference)
import functools

import jax
import jax.numpy as jnp
from jax import lax
from jax.experimental import pallas as pl
from jax.experimental.pallas import tpu as pltpu
from jax.experimental.pallas import tpu_sc as plsc

_B = 16384
_D = 32
_K = 20
_NC = 2
_NS = 16
_L = 16
_NW = _NC * _NS
_V = 1000000
_YROWS = 250016
_MAIN_BLK = 7808
_NSB = 122

_PER_W = _B // _NW
_CHUNK = 16
_NCHUNK = _PER_W // _CHUNK
_NEG_ROWS = _CHUNK * _K
_NEG_SPLIT = (128, 128, 64)

_PARAMS = pltpu.CompilerParams(needs_layout_passes=False)


def _iota():
    return lax.iota(jnp.int32, _L)


def _transpose_block(in3, out3, b, ncols):
    iot = _iota()
    bv = jnp.zeros((_L,), jnp.int32) + b
    rpat = lax.shift_right_logical(iot, 2)
    cpat = (iot & 3) << 5

    @plsc.parallel_loop(0, ncols // _L, unroll=4)
    def j_body(j):
        rowv = j * 4 + rpat
        colsv = j * _L + iot
        for d0 in range(_D):
            dvec = (d0 + iot) & (_D - 1)
            vals = plsc.load_gather(in3, [bv, dvec, colsv])
            plsc.store_scatter(out3, [bv, rowv, cpat + dvec], vals)


def _sc_transpose(ttT, ctT, ttail, ctail):
    mesh = plsc.VectorSubcoreMesh(core_axis_name="c", subcore_axis_name="s")

    @functools.partial(
        pl.kernel,
        mesh=mesh,
        out_type=(
            jax.ShapeDtypeStruct((_YROWS, 128), jnp.float32),
            jax.ShapeDtypeStruct((_YROWS, 128), jnp.float32),
        ),
        compiler_params=_PARAMS,
        scratch_types=[
            pltpu.VMEM((2, 32, 256), jnp.float32),
            pltpu.VMEM((2, 64, 128), jnp.float32),
            pltpu.SemaphoreType.DMA,
            pltpu.SemaphoreType.DMA,
            pltpu.SemaphoreType.DMA,
            pltpu.SemaphoreType.DMA,
        ],
    )
    def body(tt, ct, ttl, ctl, yt, yc, in2, out2, si0, si1, so0, so1):
        wid = lax.axis_index("s") * _NC + lax.axis_index("c")
        sin = (si0, si1)
        sout = (so0, so1)

        for src, dst in ((tt, yt), (ct, yc)):
            for b in (0, 1):
                g = wid + _NW * b
                pltpu.async_copy(
                    src.at[:, pl.ds(g * 256, 256)], in2.at[b], sin[b]
                )

            def sb_body(i2, carry, src=src, dst=dst):
                for b in (0, 1):
                    i = i2 * 2 + b
                    g = wid + _NW * i
                    pltpu.make_async_copy(
                        src.at[:, pl.ds(0, 256)], in2.at[b], sin[b]
                    ).wait()

                    @pl.when(i2 > 0)
                    def _wait_out(b=b, dst=dst):
                        pltpu.make_async_copy(
                            out2.at[b], dst.at[pl.ds(0, 64)], sout[b]
                        ).wait()

                    _transpose_block(in2, out2, b, 256)
                    pltpu.async_copy(
                        out2.at[b], dst.at[pl.ds(g * 64, 64)], sout[b]
                    )

                    @pl.when(i + 2 < _NSB)
                    def _prefetch(b=b, i=i, src=src):
                        g2 = wid + _NW * (i + 2)
                        pltpu.async_copy(
                            src.at[:, pl.ds(g2 * 256, 256)], in2.at[b], sin[b]
                        )

                return carry

            lax.fori_loop(0, _NSB // 2, sb_body, 0)
            for b in (0, 1):
                pltpu.make_async_copy(
                    out2.at[b], dst.at[pl.ds(0, 64)], sout[b]
                ).wait()

        @pl.when(wid < 4)
        def _grid_tail():
            c = _MAIN_BLK + wid
            for src, dst in ((tt, yt), (ct, yc)):
                pltpu.sync_copy(
                    src.at[:, pl.ds(c * 128, 128)],
                    in2.at[0, :, pl.ds(0, 128)],
                )
                _transpose_block(in2, out2, 0, 128)
                pltpu.sync_copy(
                    out2.at[0, pl.ds(0, 32), :], dst.at[pl.ds(c * 32, 32)]
                )

        @pl.when(wid == 4)
        def _ttail():
            pltpu.sync_copy(ttl, out2.at[1, pl.ds(0, 16), :])
            pltpu.sync_copy(out2.at[1, pl.ds(0, 16), :],
                            yt.at[pl.ds(249984, 16)])

        @pl.when(wid == 5)
        def _ctail():
            pltpu.sync_copy(ctl, out2.at[1, pl.ds(16, 16), :])
            pltpu.sync_copy(out2.at[1, pl.ds(16, 16), :],
                            yc.at[pl.ds(249984, 16)])

    return body(ttT, ctT, ttail, ctail)


def _softplus(z):
    a = jnp.abs(z)
    u = jnp.exp(-a)
    s = u / (u + 2.0)
    s2 = s * s
    p = jnp.float32(1.0 / 11.0)
    for c in (1.0 / 9.0, 1.0 / 7.0, 1.0 / 5.0, 1.0 / 3.0, 1.0):
        p = jnp.float32(c) + s2 * p
    return jnp.maximum(z, 0.0) + (2.0 * s) * p


def _clip(x):
    return jnp.minimum(jnp.maximum(x, -10.0), 10.0)


def _sc_loss(yt, yc, pos_target, pos_context, neg_flat):
    mesh = plsc.VectorSubcoreMesh(core_axis_name="c", subcore_axis_name="s")

    @functools.partial(
        pl.kernel,
        mesh=mesh,
        out_type=jax.ShapeDtypeStruct((_NW, _L), jnp.float32),
        compiler_params=_PARAMS,
        scratch_types=[
            pltpu.VMEM((_PER_W,), jnp.int32),
            pltpu.VMEM((_PER_W,), jnp.int32),
            pltpu.VMEM((_PER_W * _K,), jnp.int32),
            pltpu.VMEM((_PER_W,), jnp.int32),
            pltpu.VMEM((_PER_W,), jnp.int32),
            pltpu.VMEM((_PER_W * _K,), jnp.int32),
            pltpu.VMEM((2, _CHUNK, 128), jnp.float32),
            pltpu.VMEM((2, _CHUNK, 128), jnp.float32),
            pltpu.VMEM((2, _NEG_ROWS, 128), jnp.float32),
            pltpu.VMEM((_L,), jnp.float32),
            pltpu.SemaphoreType.DMA,
            pltpu.SemaphoreType.DMA,
        ],
    )
    def body(yt_hbm, yc_hbm, pt_hbm, pc_hbm, ni_hbm, out_hbm,
             ti, ci, ni, ti4, ci4, ni4, tv2, cv2, nv2, loss_v, sg0, sg1):
        wid = lax.axis_index("s") * _NC + lax.axis_index("c")
        iot = _iota()
        sg = (sg0, sg1)
        loss_v[...] = jnp.zeros((_L,), jnp.float32)

        pltpu.sync_copy(pt_hbm.at[pl.ds(wid * _PER_W, _PER_W)], ti)
        pltpu.sync_copy(pc_hbm.at[pl.ds(wid * _PER_W, _PER_W)], ci)
        pltpu.sync_copy(
            ni_hbm.at[pl.ds(wid * _PER_W * _K, _PER_W * _K)], ni
        )

        @plsc.parallel_loop(0, _PER_W // _L, unroll=4)
        def shift_tc(j):
            sl = pl.ds(j * _L, _L)
            ti4[sl] = lax.shift_right_logical(ti[sl], 2)
            ci4[sl] = lax.shift_right_logical(ci[sl], 2)

        @plsc.parallel_loop(0, _PER_W * _K // _L, unroll=4)
        def shift_n(j):
            sl = pl.ds(j * _L, _L)
            ni4[sl] = lax.shift_right_logical(ni[sl], 2)

        def fire(i, b):
            pltpu.async_copy(
                yt_hbm.at[ti4.at[pl.ds(i * _CHUNK, _CHUNK)]],
                tv2.at[b], sg[b],
            )
            pltpu.async_copy(
                yc_hbm.at[ci4.at[pl.ds(i * _CHUNK, _CHUNK)]],
                cv2.at[b], sg[b],
            )
            off = 0
            for sz in _NEG_SPLIT:
                pltpu.async_copy(
                    yc_hbm.at[ni4.at[pl.ds(i * _NEG_ROWS + off, sz)]],
                    nv2.at[b, pl.ds(off, sz), :], sg[b],
                )
                off += sz

        def drain(b):
            pltpu.make_async_copy(
                yt_hbm.at[pl.ds(0, _CHUNK)], tv2.at[b], sg[b]
            ).wait()
            pltpu.make_async_copy(
                yc_hbm.at[pl.ds(0, _CHUNK)], cv2.at[b], sg[b]
            ).wait()
            off = 0
            for sz in _NEG_SPLIT:
                pltpu.make_async_copy(
                    yc_hbm.at[pl.ds(0, sz)], nv2.at[b, pl.ds(off, sz), :],
                    sg[b],
                ).wait()
                off += sz

        rots = [(m + _iota()) & (_D - 1) for m in range(_D)]

        def compute(i, b):
            bv = jnp.zeros((_L,), jnp.int32) + b
            tph = (plsc.load_gather(ti, [i * _CHUNK + iot]) & 3) << 5
            cph = (plsc.load_gather(ci, [i * _CHUNK + iot]) & 3) << 5
            tcols = [
                plsc.load_gather(tv2, [bv, iot, tph + rots[m]])
                for m in range(_D)
            ]
            acc0 = jnp.zeros((_L,), jnp.float32)
            for m in range(_D):
                ccol = plsc.load_gather(cv2, [bv, iot, cph + rots[m]])
                acc0 = acc0 + tcols[m] * ccol
            total = _softplus(-_clip(acc0))

            @plsc.parallel_loop(0, _K, unroll=2, carry=total)
            def k_body(k, tot):
                rows_k = (i * _CHUNK + iot) * _K + k
                nph = (plsc.load_gather(ni, [rows_k]) & 3) << 5
                lrows = iot * _K + k
                acck = jnp.zeros((_L,), jnp.float32)
                for m in range(_D):
                    ncol = plsc.load_gather(nv2, [bv, lrows, nph + rots[m]])
                    acck = acck + tcols[m] * ncol
                return tot + _softplus(_clip(acck))

            total = k_body
            loss_v[...] = loss_v[...] + total

        fire(0, 0)

        def pair_body(i2, carry):
            for b in (0, 1):
                i = i2 * 2 + b

                @pl.when(i + 1 < _NCHUNK)
                def _prefetch(i=i, b=b):
                    fire(i + 1, 1 - b)

                drain(b)
                compute(i, b)
            return carry

        lax.fori_loop(0, _NCHUNK // 2, pair_body, 0)
        pltpu.sync_copy(loss_v, out_hbm.at[wid])

    return body(yt, yc, pos_target, pos_context, neg_flat)


def kernel(target_table, context_table, pos_target, pos_context, neg_context):
    ttail = target_table[_V - 64:].reshape(16, 128)
    ctail = context_table[_V - 64:].reshape(16, 128)
    yt, yc = _sc_transpose(target_table.T, context_table.T, ttail, ctail)
    partials = _sc_loss(yt, yc,
                        pos_target.astype(jnp.int32),
                        pos_context.astype(jnp.int32),
                        neg_context.reshape(_B * _K).astype(jnp.int32))
    return jnp.sum(partials) / jnp.float32(_B)


# speedup vs baseline: 2.1752x; 2.1752x over previous
import functools

import jax
import jax.numpy as jnp
from jax import lax
from jax.experimental import pallas as pl
from jax.experimental.pallas import tpu as pltpu
from jax.experimental.pallas import tpu_sc as plsc

_B = 16384
_D = 32
_K = 20
_NC = 2
_NS = 16
_L = 16
_NW = _NC * _NS
_V = 1000000
_YROWS = 250016
_MAIN_BLK = 7808
_SBW = 512
_NSB = 61

_PER_W = _B // _NW
_CHUNK = 16
_NCHUNK = _PER_W // _CHUNK
_NEG_ROWS = _CHUNK * _K
_NEG_SPLIT = (128, 128, 64)

_PARAMS = pltpu.CompilerParams(needs_layout_passes=False)


def _iota():
    return lax.iota(jnp.int32, _L)


def _transpose_block(in3, out3, b, ncols):
    iot = _iota()
    bv = jnp.zeros((_L,), jnp.int32) + b
    rpat = lax.shift_right_logical(iot, 2)
    cpat = (iot & 3) << 5

    @plsc.parallel_loop(0, ncols // _L, unroll=2)
    def j_body(j):
        rowv = j * 4 + rpat
        colsv = j * _L + iot
        for d0 in range(_D):
            dvec = (d0 + iot) & (_D - 1)
            vals = plsc.load_gather(in3, [bv, dvec, colsv])
            plsc.store_scatter(out3, [bv, rowv, cpat + dvec], vals)


def _sc_transpose(ttT, ctT, ttail, ctail):
    mesh = plsc.VectorSubcoreMesh(core_axis_name="c", subcore_axis_name="s")

    @functools.partial(
        pl.kernel,
        mesh=mesh,
        out_type=(
            jax.ShapeDtypeStruct((_YROWS, 128), jnp.float32),
            jax.ShapeDtypeStruct((_YROWS, 128), jnp.float32),
        ),
        compiler_params=_PARAMS,
        scratch_types=[
            pltpu.VMEM((2, 32, _SBW), jnp.float32),
            pltpu.VMEM((2, _SBW // 4, 128), jnp.float32),
            pltpu.SemaphoreType.DMA,
            pltpu.SemaphoreType.DMA,
            pltpu.SemaphoreType.DMA,
            pltpu.SemaphoreType.DMA,
        ],
    )
    def body(tt, ct, ttl, ctl, yt, yc, in2, out2, si0, si1, so0, so1):
        wid = lax.axis_index("s") * _NC + lax.axis_index("c")
        sin = (si0, si1)
        sout = (so0, so1)

        yrows = _SBW // 4

        for src, dst in ((tt, yt), (ct, yc)):

            def issue_in(i, b, src=src):
                g = wid + _NW * i
                pltpu.async_copy(
                    src.at[:, pl.ds(g * _SBW, _SBW)], in2.at[b], sin[b]
                )

            def wait_in(b, src=src):
                pltpu.make_async_copy(
                    src.at[:, pl.ds(0, _SBW)], in2.at[b], sin[b]
                ).wait()

            def issue_out(i, b, dst=dst):
                g = wid + _NW * i
                pltpu.async_copy(
                    out2.at[b], dst.at[pl.ds(g * yrows, yrows)], sout[b]
                )

            def wait_out(b, dst=dst):
                pltpu.make_async_copy(
                    out2.at[b], dst.at[pl.ds(0, yrows)], sout[b]
                ).wait()

            for b in (0, 1):
                issue_in(b, b)

            def sb_body(i2, carry):
                for b in (0, 1):
                    i = i2 * 2 + b
                    wait_in(b)

                    @pl.when(i2 > 0)
                    def _wait_out(b=b):
                        wait_out(b)

                    _transpose_block(in2, out2, b, _SBW)
                    issue_out(i, b)

                    @pl.when(i + 2 < _NSB)
                    def _prefetch(b=b, i=i):
                        issue_in(i + 2, b)

                return carry

            lax.fori_loop(0, (_NSB - 1) // 2, sb_body, 0)
            wait_in(0)
            wait_out(0)
            _transpose_block(in2, out2, 0, _SBW)
            issue_out(_NSB - 1, 0)
            for b in (0, 1):
                wait_out(b)

        @pl.when(wid < 4)
        def _grid_tail():
            c = _MAIN_BLK + wid
            for src, dst in ((tt, yt), (ct, yc)):
                pltpu.sync_copy(
                    src.at[:, pl.ds(c * 128, 128)],
                    in2.at[0, :, pl.ds(0, 128)],
                )
                _transpose_block(in2, out2, 0, 128)
                pltpu.sync_copy(
                    out2.at[0, pl.ds(0, 32), :], dst.at[pl.ds(c * 32, 32)]
                )

        @pl.when(wid == 4)
        def _ttail():
            pltpu.sync_copy(ttl, out2.at[1, pl.ds(0, 16), :])
            pltpu.sync_copy(out2.at[1, pl.ds(0, 16), :],
                            yt.at[pl.ds(249984, 16)])

        @pl.when(wid == 5)
        def _ctail():
            pltpu.sync_copy(ctl, out2.at[1, pl.ds(16, 16), :])
            pltpu.sync_copy(out2.at[1, pl.ds(16, 16), :],
                            yc.at[pl.ds(249984, 16)])

    return body(ttT, ctT, ttail, ctail)


def _softplus(z):
    a = jnp.abs(z)
    u = jnp.exp(-a)
    s = u / (u + 2.0)
    s2 = s * s
    p = jnp.float32(1.0 / 11.0)
    for c in (1.0 / 9.0, 1.0 / 7.0, 1.0 / 5.0, 1.0 / 3.0, 1.0):
        p = jnp.float32(c) + s2 * p
    return jnp.maximum(z, 0.0) + (2.0 * s) * p


def _clip(x):
    return jnp.minimum(jnp.maximum(x, -10.0), 10.0)


def _sc_loss(yt, yc, pos_target, pos_context, neg_flat):
    mesh = plsc.VectorSubcoreMesh(core_axis_name="c", subcore_axis_name="s")

    @functools.partial(
        pl.kernel,
        mesh=mesh,
        out_type=jax.ShapeDtypeStruct((_NW, _L), jnp.float32),
        compiler_params=_PARAMS,
        scratch_types=[
            pltpu.VMEM((_PER_W,), jnp.int32),
            pltpu.VMEM((_PER_W,), jnp.int32),
            pltpu.VMEM((_PER_W * _K,), jnp.int32),
            pltpu.VMEM((_PER_W,), jnp.int32),
            pltpu.VMEM((_PER_W,), jnp.int32),
            pltpu.VMEM((_PER_W * _K,), jnp.int32),
            pltpu.VMEM((2, _CHUNK, 128), jnp.float32),
            pltpu.VMEM((2, _CHUNK, 128), jnp.float32),
            pltpu.VMEM((2, _NEG_ROWS, 128), jnp.float32),
            pltpu.VMEM((_L,), jnp.float32),
            pltpu.SemaphoreType.DMA,
            pltpu.SemaphoreType.DMA,
        ],
    )
    def body(yt_hbm, yc_hbm, pt_hbm, pc_hbm, ni_hbm, out_hbm,
             ti, ci, ni, ti4, ci4, ni4, tv2, cv2, nv2, loss_v, sg0, sg1):
        wid = lax.axis_index("s") * _NC + lax.axis_index("c")
        iot = _iota()
        sg = (sg0, sg1)
        loss_v[...] = jnp.zeros((_L,), jnp.float32)

        pltpu.sync_copy(pt_hbm.at[pl.ds(wid * _PER_W, _PER_W)], ti)
        pltpu.sync_copy(pc_hbm.at[pl.ds(wid * _PER_W, _PER_W)], ci)
        pltpu.sync_copy(
            ni_hbm.at[pl.ds(wid * _PER_W * _K, _PER_W * _K)], ni
        )

        @plsc.parallel_loop(0, _PER_W // _L, unroll=4)
        def shift_tc(j):
            sl = pl.ds(j * _L, _L)
            ti4[sl] = lax.shift_right_logical(ti[sl], 2)
            ci4[sl] = lax.shift_right_logical(ci[sl], 2)

        @plsc.parallel_loop(0, _PER_W * _K // _L, unroll=4)
        def shift_n(j):
            sl = pl.ds(j * _L, _L)
            ni4[sl] = lax.shift_right_logical(ni[sl], 2)

        def fire(i, b):
            pltpu.async_copy(
                yt_hbm.at[ti4.at[pl.ds(i * _CHUNK, _CHUNK)]],
                tv2.at[b], sg[b],
            )
            pltpu.async_copy(
                yc_hbm.at[ci4.at[pl.ds(i * _CHUNK, _CHUNK)]],
                cv2.at[b], sg[b],
            )
            off = 0
            for sz in _NEG_SPLIT:
                pltpu.async_copy(
                    yc_hbm.at[ni4.at[pl.ds(i * _NEG_ROWS + off, sz)]],
                    nv2.at[b, pl.ds(off, sz), :], sg[b],
                )
                off += sz

        def drain(b):
            pltpu.make_async_copy(
                yt_hbm.at[pl.ds(0, _CHUNK)], tv2.at[b], sg[b]
            ).wait()
            pltpu.make_async_copy(
                yc_hbm.at[pl.ds(0, _CHUNK)], cv2.at[b], sg[b]
            ).wait()
            off = 0
            for sz in _NEG_SPLIT:
                pltpu.make_async_copy(
                    yc_hbm.at[pl.ds(0, sz)], nv2.at[b, pl.ds(off, sz), :],
                    sg[b],
                ).wait()
                off += sz

        rots = [(m + _iota()) & (_D - 1) for m in range(_D)]

        def compute(i, b):
            bv = jnp.zeros((_L,), jnp.int32) + b
            tph = (plsc.load_gather(ti, [i * _CHUNK + iot]) & 3) << 5
            cph = (plsc.load_gather(ci, [i * _CHUNK + iot]) & 3) << 5
            tcols = [
                plsc.load_gather(tv2, [bv, iot, tph + rots[m]])
                for m in range(_D)
            ]
            acc0 = jnp.zeros((_L,), jnp.float32)
            for m in range(_D):
                ccol = plsc.load_gather(cv2, [bv, iot, cph + rots[m]])
                acc0 = acc0 + tcols[m] * ccol
            total = _softplus(-_clip(acc0))

            @plsc.parallel_loop(0, _K, unroll=2, carry=total)
            def k_body(k, tot):
                rows_k = (i * _CHUNK + iot) * _K + k
                nph = (plsc.load_gather(ni, [rows_k]) & 3) << 5
                lrows = iot * _K + k
                acck = jnp.zeros((_L,), jnp.float32)
                for m in range(_D):
                    ncol = plsc.load_gather(nv2, [bv, lrows, nph + rots[m]])
                    acck = acck + tcols[m] * ncol
                return tot + _softplus(_clip(acck))

            total = k_body
            loss_v[...] = loss_v[...] + total

        fire(0, 0)

        def pair_body(i2, carry):
            for b in (0, 1):
                i = i2 * 2 + b

                @pl.when(i + 1 < _NCHUNK)
                def _prefetch(i=i, b=b):
                    fire(i + 1, 1 - b)

                drain(b)
                compute(i, b)
            return carry

        lax.fori_loop(0, _NCHUNK // 2, pair_body, 0)
        pltpu.sync_copy(loss_v, out_hbm.at[wid])

    return body(yt, yc, pos_target, pos_context, neg_flat)


def kernel(target_table, context_table, pos_target, pos_context, neg_context):
    ttail = target_table[_V - 64:].reshape(16, 128)
    ctail = context_table[_V - 64:].reshape(16, 128)
    yt, yc = _sc_transpose(target_table.T, context_table.T, ttail, ctail)
    partials = _sc_loss(yt, yc,
                        pos_target.astype(jnp.int32),
                        pos_context.astype(jnp.int32),
                        neg_context.reshape(_B * _K).astype(jnp.int32))
    return jnp.sum(partials) / jnp.float32(_B)
